# baseline (speedup 1.0000x reference)
import functools

import jax
import jax.numpy as jnp
from jax import lax
from jax.experimental import pallas as pl
from jax.experimental.pallas import tpu as pltpu
from jax.experimental.pallas import tpu_sc as plsc

B_ROWS = 4096
SEQ = 200
D_MODEL = 64
VOCAB = 1000000
SCALE = float(D_MODEL) ** 0.5
LANES = 16

NC, NS = 2, 16
NW = NC * NS
BW = B_ROWS // NW
NBUF = 4
NOBUF = 2
ROUNDS = SEQ // NBUF
CT, CI = D_MODEL // 8, 8
BT = B_ROWS // 128
KG = BW // LANES
TBLK = 1024


def _tc_prep_body(tt_ref, out_ref):
    blk = (tt_ref[...] * SCALE).T
    out_ref[...] = jnp.concatenate([blk, blk], axis=1)


_prep = pl.pallas_call(
    _tc_prep_body,
    grid=((VOCAB + TBLK - 1) // TBLK,),
    in_specs=[pl.BlockSpec((D_MODEL, TBLK), lambda i: (0, i))],
    out_specs=pl.BlockSpec((TBLK, 128), lambda i: (i, 0)),
    out_shape=jax.ShapeDtypeStruct((VOCAB, 128), jnp.float32),
)


def _tec_body(xt_hbm, tdup_hbm, out_hbm, *sc):
    idx_v = sc[0]
    gbuf = sc[1:1 + NBUF]
    obuf = sc[1 + NBUF:1 + NBUF + NOBUF]
    gsem = sc[1 + NBUF + NOBUF:1 + 2 * NBUF + NOBUF]
    osem = sc[1 + 2 * NBUF + NOBUF:]

    wid = lax.axis_index("c") * NS + lax.axis_index("s")
    b0 = wid * BW

    pltpu.sync_copy(xt_hbm.at[:, pl.ds(b0, BW)], idx_v)

    row16 = [jax.lax.iota(jnp.int32, LANES) + (k * LANES) for k in range(KG)]

    def start_gather(b, t):
        pltpu.async_copy(tdup_hbm.at[idx_v.at[t]], gbuf[b], gsem[b])

    def wait_gather(b):
        pltpu.make_async_copy(tdup_hbm.at[idx_v.at[0]], gbuf[b], gsem[b]).wait()

    def start_out(ob, t):
        pltpu.async_copy(obuf[ob], out_hbm.at[t, :, wid], osem[ob])

    def wait_out(ob):
        pltpu.make_async_copy(obuf[ob], out_hbm.at[0, :, wid], osem[ob]).wait()

    def transpose_t(b, ob):
        gb, o = gbuf[b], obuf[ob]

        def body_fn(c):
            ct = c // CI
            ci = c % CI
            cvec = jnp.zeros((LANES,), jnp.int32) + c
            for k in range(KG):
                vals = plsc.load_gather(gb, [row16[k], cvec])
                o[ct, ci, pl.ds(k * LANES, LANES)] = vals

        plsc.parallel_loop(0, D_MODEL, 1, unroll=4)(body_fn)

    for b in range(NBUF):
        start_gather(b, b)

    for b in range(NBUF):
        wait_gather(b)
        ob = b % NOBUF
        if b >= NOBUF:
            wait_out(ob)
        transpose_t(b, ob)
        start_gather(b, b + NBUF)
        start_out(ob, b)

    def round_body(ro, carry):
        for b in range(NBUF):
            t = ro * NBUF + b
            wait_gather(b)
            ob = b % NOBUF
            wait_out(ob)
            transpose_t(b, ob)
            start_gather(b, t + NBUF)
            start_out(ob, t)
        return carry

    lax.fori_loop(1, ROUNDS - 1, round_body, 0)

    for b in range(NBUF):
        t = (ROUNDS - 1) * NBUF + b
        wait_gather(b)
        ob = b % NOBUF
        wait_out(ob)
        transpose_t(b, ob)
        start_out(ob, t)

    for ob in range(NOBUF):
        wait_out(ob)


_emb = functools.partial(
    pl.kernel,
    out_type=jax.ShapeDtypeStruct((SEQ, CT, BT, CI, 128), jnp.float32),
    mesh=plsc.VectorSubcoreMesh(core_axis_name="c", subcore_axis_name="s"),
    scratch_types=(
        [pltpu.VMEM((SEQ, BW), jnp.int32)]
        + [pltpu.VMEM((BW, 128), jnp.float32) for _ in range(NBUF)]
        + [pltpu.VMEM((CT, CI, 128), jnp.float32) for _ in range(NOBUF)]
        + [pltpu.SemaphoreType.DMA for _ in range(NBUF + NOBUF)]
    ),
    compiler_params=pltpu.CompilerParams(
        use_tc_tiling_on_sc=False, needs_layout_passes=False),
)(_tec_body)


def kernel(x, table):
    tdup = _prep(table.T)
    out5 = _emb(x.T.astype(jnp.int32), tdup)
    return jnp.transpose(out5, (2, 4, 0, 1, 3)).reshape(B_ROWS, SEQ, D_MODEL)


# speedup vs baseline: 1.3239x; 1.3239x over previous
import functools

import jax
import jax.numpy as jnp
from jax import lax
from jax.experimental import pallas as pl
from jax.experimental.pallas import tpu as pltpu
from jax.experimental.pallas import tpu_sc as plsc

B_ROWS = 4096
SEQ = 200
D_MODEL = 64
VOCAB = 1000000
SCALE = float(D_MODEL) ** 0.5
LANES = 16

NC, NS = 2, 16
NW = NC * NS
BW = B_ROWS // NW
NBUF = 4
NOBUF = 2
ROUNDS = SEQ // NBUF
CT, CI = D_MODEL // 8, 8
BT = B_ROWS // 128
KG = BW // LANES
TBLK = 4096


def _tc_prep_body(tt_ref, out_ref):
    blk = (tt_ref[...] * SCALE).T
    out_ref[...] = jnp.concatenate([blk, blk], axis=1)


_prep = pl.pallas_call(
    _tc_prep_body,
    grid=((VOCAB + TBLK - 1) // TBLK,),
    in_specs=[pl.BlockSpec((D_MODEL, TBLK), lambda i: (0, i))],
    out_specs=pl.BlockSpec((TBLK, 128), lambda i: (i, 0)),
    out_shape=jax.ShapeDtypeStruct((VOCAB, 128), jnp.float32),
)


def _tec_body(xt_hbm, tdup_hbm, out_hbm, *sc):
    idx_v = sc[0]
    gbuf = sc[1:1 + NBUF]
    obuf = sc[1 + NBUF:1 + NBUF + NOBUF]
    gsem = sc[1 + NBUF + NOBUF:1 + 2 * NBUF + NOBUF]
    osem = sc[1 + 2 * NBUF + NOBUF:]

    wid = lax.axis_index("c") * NS + lax.axis_index("s")
    b0 = wid * BW

    pltpu.sync_copy(xt_hbm.at[:, pl.ds(b0, BW)], idx_v)

    row16 = [jax.lax.iota(jnp.int32, LANES) + (k * LANES) for k in range(KG)]

    def start_gather(b, t):
        pltpu.async_copy(tdup_hbm.at[idx_v.at[t]], gbuf[b], gsem[b])

    def wait_gather(b):
        pltpu.make_async_copy(tdup_hbm.at[idx_v.at[0]], gbuf[b], gsem[b]).wait()

    def start_out(ob, t):
        pltpu.async_copy(obuf[ob], out_hbm.at[t, :, wid], osem[ob])

    def wait_out(ob):
        pltpu.make_async_copy(obuf[ob], out_hbm.at[0, :, wid], osem[ob]).wait()

    def transpose_t(b, ob):
        gb, o = gbuf[b], obuf[ob]

        def body_fn(c):
            ct = c // CI
            ci = c % CI
            cvec = jnp.zeros((LANES,), jnp.int32) + c
            for k in range(KG):
                vals = plsc.load_gather(gb, [row16[k], cvec])
                o[ct, ci, pl.ds(k * LANES, LANES)] = vals

        plsc.parallel_loop(0, D_MODEL, 1, unroll=8)(body_fn)

    for b in range(NBUF):
        start_gather(b, b)

    for b in range(NBUF):
        wait_gather(b)
        ob = b % NOBUF
        if b >= NOBUF:
            wait_out(ob)
        transpose_t(b, ob)
        start_gather(b, b + NBUF)
        start_out(ob, b)

    def round_body(ro, carry):
        for b in range(NBUF):
            t = ro * NBUF + b
            wait_gather(b)
            ob = b % NOBUF
            wait_out(ob)
            transpose_t(b, ob)
            start_gather(b, t + NBUF)
            start_out(ob, t)
        return carry

    lax.fori_loop(1, ROUNDS - 1, round_body, 0)

    for b in range(NBUF):
        t = (ROUNDS - 1) * NBUF + b
        wait_gather(b)
        ob = b % NOBUF
        wait_out(ob)
        transpose_t(b, ob)
        start_out(ob, t)

    for ob in range(NOBUF):
        wait_out(ob)


_emb = functools.partial(
    pl.kernel,
    out_type=jax.ShapeDtypeStruct((SEQ, CT, BT, CI, 128), jnp.float32),
    mesh=plsc.VectorSubcoreMesh(core_axis_name="c", subcore_axis_name="s"),
    scratch_types=(
        [pltpu.VMEM((SEQ, BW), jnp.int32)]
        + [pltpu.VMEM((BW, 128), jnp.float32) for _ in range(NBUF)]
        + [pltpu.VMEM((CT, CI, 128), jnp.float32) for _ in range(NOBUF)]
        + [pltpu.SemaphoreType.DMA for _ in range(NBUF + NOBUF)]
    ),
    compiler_params=pltpu.CompilerParams(
        use_tc_tiling_on_sc=False, needs_layout_passes=False),
)(_tec_body)


def kernel(x, table):
    tdup = _prep(table.T)
    out5 = _emb(x.T.astype(jnp.int32), tdup)
    return jnp.transpose(out5, (2, 4, 0, 1, 3)).reshape(B_ROWS, SEQ, D_MODEL)


# speedup vs baseline: 2.4844x; 1.8766x over previous
import functools

import jax
import jax.numpy as jnp
from jax import lax
from jax.experimental import pallas as pl
from jax.experimental.pallas import tpu as pltpu
from jax.experimental.pallas import tpu_sc as plsc

B_ROWS = 4096
SEQ = 200
D_MODEL = 64
VOCAB = 1000000
SCALE = float(D_MODEL) ** 0.5
LANES = 16

NC, NS = 2, 16
NW = NC * NS
BW = B_ROWS // NW
NBUF = 4
NOBUF = 2
ROUNDS = SEQ // NBUF
CT, CI = D_MODEL // 8, 8
BT = B_ROWS // 128
KG = BW // LANES
TBLK = 4096


def _tc_prep_body(tt_ref, out_ref):
    blk = (tt_ref[...] * SCALE).T
    out_ref[:, 0:D_MODEL] = blk


_prep = pl.pallas_call(
    _tc_prep_body,
    grid=((VOCAB + TBLK - 1) // TBLK,),
    in_specs=[pl.BlockSpec((D_MODEL, TBLK), lambda i: (0, i))],
    out_specs=pl.BlockSpec((TBLK, 128), lambda i: (i, 0)),
    out_shape=jax.ShapeDtypeStruct((VOCAB, 128), jnp.float32),
)


def _tec_body(xt_hbm, tdup_hbm, out_hbm, *sc):
    idx_v = sc[0]
    gbuf = sc[1:1 + NBUF]
    obuf = sc[1 + NBUF:1 + NBUF + NOBUF]
    gsem = sc[1 + NBUF + NOBUF:1 + 2 * NBUF + NOBUF]
    osem = sc[1 + 2 * NBUF + NOBUF:]

    wid = lax.axis_index("c") * NS + lax.axis_index("s")
    b0 = wid * BW

    pltpu.sync_copy(xt_hbm.at[:, pl.ds(b0, BW)], idx_v)

    row16 = [jax.lax.iota(jnp.int32, LANES) + (k * LANES) for k in range(KG)]

    def start_gather(b, t):
        pltpu.async_copy(tdup_hbm.at[idx_v.at[t]], gbuf[b], gsem[b])

    def wait_gather(b):
        pltpu.make_async_copy(tdup_hbm.at[idx_v.at[0]], gbuf[b], gsem[b]).wait()

    def start_out(ob, t):
        pltpu.async_copy(obuf[ob], out_hbm.at[t, :, wid], osem[ob])

    def wait_out(ob):
        pltpu.make_async_copy(obuf[ob], out_hbm.at[0, :, wid], osem[ob]).wait()

    lane = jax.lax.iota(jnp.int32, LANES)

    def transpose_t(b, ob):
        gb, o = gbuf[b], obuf[ob]

        def body_fn(c):
            colv = jax.lax.bitwise_and(lane + c, D_MODEL - 1)
            cth = jax.lax.shift_right_logical(colv, 3)
            cil = jax.lax.bitwise_and(colv, CI - 1)
            for k in range(KG):
                vals = plsc.load_gather(gb, [row16[k], colv])
                plsc.store_scatter(o, [cth, cil, row16[k]], vals)

        plsc.parallel_loop(0, D_MODEL, 1, unroll=8)(body_fn)

    for b in range(NBUF):
        start_gather(b, b)

    for b in range(NBUF):
        wait_gather(b)
        ob = b % NOBUF
        if b >= NOBUF:
            wait_out(ob)
        transpose_t(b, ob)
        start_gather(b, b + NBUF)
        start_out(ob, b)

    def round_body(ro, carry):
        for b in range(NBUF):
            t = ro * NBUF + b
            wait_gather(b)
            ob = b % NOBUF
            wait_out(ob)
            transpose_t(b, ob)
            start_gather(b, t + NBUF)
            start_out(ob, t)
        return carry

    lax.fori_loop(1, ROUNDS - 1, round_body, 0)

    for b in range(NBUF):
        t = (ROUNDS - 1) * NBUF + b
        wait_gather(b)
        ob = b % NOBUF
        wait_out(ob)
        transpose_t(b, ob)
        start_out(ob, t)

    for ob in range(NOBUF):
        wait_out(ob)


_emb = functools.partial(
    pl.kernel,
    out_type=jax.ShapeDtypeStruct((SEQ, CT, BT, CI, 128), jnp.float32),
    mesh=plsc.VectorSubcoreMesh(core_axis_name="c", subcore_axis_name="s"),
    scratch_types=(
        [pltpu.VMEM((SEQ, BW), jnp.int32)]
        + [pltpu.VMEM((BW, 128), jnp.float32) for _ in range(NBUF)]
        + [pltpu.VMEM((CT, CI, 128), jnp.float32) for _ in range(NOBUF)]
        + [pltpu.SemaphoreType.DMA for _ in range(NBUF + NOBUF)]
    ),
    compiler_params=pltpu.CompilerParams(
        use_tc_tiling_on_sc=False, needs_layout_passes=False),
)(_tec_body)


def kernel(x, table):
    tdup = _prep(table.T)
    out5 = _emb(x.T.astype(jnp.int32), tdup)
    return jnp.transpose(out5, (2, 4, 0, 1, 3)).reshape(B_ROWS, SEQ, D_MODEL)
